# baseline (speedup 1.0000x reference)
import jax
import jax.numpy as jnp
from jax.experimental import pallas as pl
from jax.experimental.pallas import tpu as pltpu

_N = 1_000_000
_C = 1024
_NFULL = _N // _C
_LAST = _N - _C


def _threefry_bits(idx_u32):
    ks0 = jnp.uint32(0)
    ks1 = jnp.uint32(42)
    ks2 = ks0 ^ ks1 ^ jnp.uint32(0x1BD11BDA)
    ks = (ks0, ks1, ks2)
    r0 = (13, 15, 26, 6)
    r1 = (17, 29, 16, 24)

    x0 = jnp.broadcast_to(ks0, idx_u32.shape)
    x1 = idx_u32 + ks1

    def rounds(x0, x1, rots):
        for r in rots:
            x0 = x0 + x1
            x1 = (x1 << jnp.uint32(r)) | (x1 >> jnp.uint32(32 - r))
            x1 = x0 ^ x1
        return x0, x1

    for i, rots in enumerate((r0, r1, r0, r1, r0)):
        x0, x1 = rounds(x0, x1, rots)
        x0 = x0 + ks[(i + 1) % 3]
        x1 = x1 + ks[(i + 2) % 3] + jnp.uint32(i + 1)
    return x0 ^ x1


def _gumbel(gidx):
    bits = _threefry_bits(gidx.astype(jnp.uint32))
    fbits = (bits >> jnp.uint32(9)) | jnp.uint32(0x3F800000)
    f = jax.lax.bitcast_convert_type(fbits, jnp.float32)
    eps = jnp.float32(1e-10)
    u = jnp.maximum(eps, (f - jnp.float32(1.0)) + eps)
    return -jnp.log(-jnp.log(u))


def _body(l_ref, out_ref):
    row = jax.lax.broadcasted_iota(jnp.int32, (8, 128), 0)
    col = jax.lax.broadcasted_iota(jnp.int32, (8, 128), 1)
    rc = row * 128 + col

    zm = jnp.full((8, 128), -jnp.inf, jnp.float32)
    im = jnp.zeros((8, 128), jnp.int32)
    bases = [k * _C for k in range(_NFULL)] + [_LAST]
    for base in bases:
        v = jnp.reshape(l_ref[pl.ds(base, _C)], (8, 128))
        z = v + _gumbel(base + rc)
        upd = z > zm
        zm = jnp.where(upd, z, zm)
        im = jnp.where(upd, base + rc, im)

    m = jnp.max(zm)
    cand = jnp.where(zm == m, im, jnp.int32(0x7FFFFFFF))
    out_ref[0] = jnp.min(cand)


def kernel(logits):
    out = pl.pallas_call(
        _body,
        out_specs=pl.BlockSpec(memory_space=pltpu.SMEM),
        out_shape=jax.ShapeDtypeStruct((1,), jnp.int32),
    )(logits)
    return out[0]
